# baseline (speedup 1.0000x reference)
import functools

import jax
import jax.numpy as jnp
from jax import lax
from jax.experimental import pallas as pl
from jax.experimental.pallas import tpu as pltpu
from jax.experimental.pallas import tpu_sc as plsc

BATCH = 16384
DIM = 64
NUM_CORES = 2
NUM_SUBCORES = 16
NUM_WORKERS = NUM_CORES * NUM_SUBCORES
B_PER_W = BATCH // NUM_WORKERS
CHUNK = 128
NCHUNK = B_PER_W // CHUNK


def _make_gather():
    mesh = plsc.VectorSubcoreMesh(core_axis_name="c", subcore_axis_name="s")

    @functools.partial(
        pl.kernel,
        mesh=mesh,
        out_type=jax.ShapeDtypeStruct((BATCH, DIM), jnp.float32),
        scratch_types=[
            pltpu.VMEM((NCHUNK, CHUNK), jnp.int32),
            pltpu.VMEM((B_PER_W, DIM), jnp.float32),
            pltpu.SemaphoreType.DMA,
        ],
        compiler_params=pltpu.CompilerParams(use_tc_tiling_on_sc=False),
    )
    def gather_kernel(idx_hbm, table_hbm, out_hbm, idx_v, rows_v, sem):
        wid = lax.axis_index("s") * NUM_CORES + lax.axis_index("c")
        base = wid * B_PER_W
        pltpu.sync_copy(idx_hbm.at[wid], idx_v)
        copies = [
            pltpu.async_copy(
                table_hbm.at[idx_v.at[j]],
                rows_v.at[pl.ds(j * CHUNK, CHUNK)],
                sem,
            )
            for j in range(NCHUNK)
        ]
        for c in copies:
            c.wait()
        pltpu.sync_copy(rows_v, out_hbm.at[pl.ds(base, B_PER_W)])

    return gather_kernel


_gather = _make_gather()


def kernel(x, embedding_location):
    idx = x[:, 0].astype(jnp.int32).reshape(NUM_WORKERS, NCHUNK, CHUNK)
    return _gather(idx, embedding_location)


# speedup vs baseline: 1.0013x; 1.0013x over previous
import functools

import jax
import jax.numpy as jnp
from jax import lax
from jax.experimental import pallas as pl
from jax.experimental.pallas import tpu as pltpu
from jax.experimental.pallas import tpu_sc as plsc

BATCH = 16384
DIM = 64
NUM_CORES = 2
NUM_SUBCORES = 16
NUM_WORKERS = NUM_CORES * NUM_SUBCORES
B_PER_W = BATCH // NUM_WORKERS
CHUNK = 128
NCHUNK = B_PER_W // CHUNK


def _make_gather():
    mesh = plsc.VectorSubcoreMesh(core_axis_name="c", subcore_axis_name="s")

    @functools.partial(
        pl.kernel,
        mesh=mesh,
        out_type=jax.ShapeDtypeStruct((BATCH, DIM), jnp.float32),
        scratch_types=[
            pltpu.VMEM((B_PER_W,), jnp.int32),
            pltpu.VMEM((B_PER_W, DIM), jnp.float32),
            pltpu.SemaphoreType.DMA,
        ],
        compiler_params=pltpu.CompilerParams(use_tc_tiling_on_sc=False),
    )
    def gather_kernel(idx_hbm, table_hbm, out_hbm, idx_v, rows_v, sem):
        wid = lax.axis_index("s") * NUM_CORES + lax.axis_index("c")
        base = wid * B_PER_W
        pltpu.sync_copy(idx_hbm.at[pl.ds(base, B_PER_W)], idx_v)
        copies = [
            pltpu.async_copy(
                table_hbm.at[idx_v.at[pl.ds(j * CHUNK, CHUNK)]],
                rows_v.at[pl.ds(j * CHUNK, CHUNK)],
                sem,
            )
            for j in range(NCHUNK)
        ]
        for c in copies:
            c.wait()
        pltpu.sync_copy(rows_v, out_hbm.at[pl.ds(base, B_PER_W)])

    return gather_kernel


_gather = _make_gather()


def kernel(x, embedding_location):
    idx = x[:, 0].astype(jnp.int32)
    return _gather(idx, embedding_location)


# speedup vs baseline: 1.0532x; 1.0519x over previous
import functools

import jax
import jax.numpy as jnp
from jax import lax
from jax.experimental import pallas as pl
from jax.experimental.pallas import tpu as pltpu
from jax.experimental.pallas import tpu_sc as plsc

BATCH = 16384
DIM = 64
NUM_CORES = 2
NUM_SUBCORES = 16
NUM_WORKERS = NUM_CORES * NUM_SUBCORES

NUM_LOC = 1000000
TAIL_START = (NUM_LOC // 128) * 128
WIN = 512
TC_PER_W = 245
N_WIN = 62
MAX_START_TC = (TAIL_START // 128) - 4
BUCKET_TC = TC_PER_W + 3

DUMP_ROW = BATCH
OUT_ROWS = BATCH + 128
FLUSH_AT = 108


def _make_gather():
    mesh = plsc.VectorSubcoreMesh(core_axis_name="c", subcore_axis_name="s")

    @functools.partial(
        pl.kernel,
        mesh=mesh,
        out_type=jax.ShapeDtypeStruct((OUT_ROWS, 128), jnp.float32),
        scratch_types=[
            pltpu.VMEM((BATCH,), jnp.int32),
            pltpu.VMEM((BATCH + 16,), jnp.int32),
            pltpu.VMEM((BATCH + 16,), jnp.int32),
            pltpu.VMEM((DIM, WIN), jnp.float32),
            pltpu.VMEM((DIM, 128), jnp.float32),
            pltpu.VMEM((128, 128), jnp.float32),
            pltpu.VMEM((1, 128), jnp.int32),
            pltpu.SemaphoreType.DMA,
        ],
        compiler_params=pltpu.CompilerParams(needs_layout_passes=False),
    )
    def gather_kernel(idx_hbm, table_hbm, tail_hbm, out_hbm,
                      idx_v, my_idx, my_pos, win_v, tail_v, stage, posrow,
                      sem):
        wid = lax.axis_index("s") * NUM_CORES + lax.axis_index("c")
        lanes = jnp.arange(16, dtype=jnp.int32)

        pltpu.sync_copy(idx_hbm, idx_v)
        pltpu.sync_copy(tail_hbm, tail_v)

        lo_t = wid * (TC_PER_W * 128)
        hi_t = jnp.minimum((wid * TC_PER_W + BUCKET_TC + 1) * 128, NUM_LOC)

        SENTINEL = jnp.int32(0x7FFFFFFF)

        @pl.loop(0, BATCH // 16, init_carry=jnp.int32(0))
        def build(g, off):
            v = idx_v[pl.ds(g * 16, 16)]
            m = (v >= lo_t) & (v < hi_t)
            key = jnp.where(m, v, SENTINEL)
            sk, sp = plsc.sort_key_val(key, lanes + g * 16)
            my_idx[pl.ds(off, 16)] = sk
            my_pos[pl.ds(off, 16)] = sp
            return off + jnp.sum(m.astype(jnp.int32))

        n_my = build
        n_grp = (n_my + 15) >> 4

        def reset_posrow():
            for k in range(8):
                posrow[0, pl.ds(k * 16, 16)] = jnp.full(
                    (16,), DUMP_ROW, jnp.int32)

        reset_posrow()

        def flush():
            pltpu.async_copy(stage, out_hbm.at[posrow.at[0]], sem).wait()
            reset_posrow()

        def extract(win_ref, lo, width, fill):
            @pl.loop(0, n_grp, init_carry=fill)
            def scan(k, fill):
                v = my_idx[pl.ds(k * 16, 16)]
                p = my_pos[pl.ds(k * 16, 16)]
                m = (v >= lo) & (v < lo + width) & (lanes + k * 16 < n_my)
                n = jnp.sum(m.astype(jnp.int32))

                @pl.when(n > 0)
                def _():
                    il = jnp.where(m, v - lo, 0)
                    raw = fill + plsc.cumsum(m.astype(jnp.int32)) - 1
                    slots = jnp.where(m, raw, 127)
                    p_safe = jnp.where(m, p, DUMP_ROW)
                    for c in range(DIM):
                        cc = jnp.full((16,), c, jnp.int32)
                        vals = plsc.load_gather(win_ref, [cc, il])
                        plsc.store_scatter(stage, [slots, cc], vals)
                    plsc.store_scatter(posrow,
                                       [jnp.zeros((16,), jnp.int32), slots],
                                       p_safe)

                fill = fill + n

                @pl.when(fill >= FLUSH_AT)
                def _():
                    flush()

                return jnp.where(fill >= FLUSH_AT, 0, fill)

            return scan

        @pl.loop(0, N_WIN, init_carry=jnp.int32(0))
        def windows(w, fill):
            start_tc = jnp.minimum(wid * TC_PER_W + w * 4, MAX_START_TC)
            lo = start_tc * 128
            pltpu.sync_copy(table_hbm.at[:, pl.ds(lo, WIN)], win_v)
            return extract(win_v, lo, WIN, fill)

        fill = extract(tail_v, jnp.int32(TAIL_START),
                       NUM_LOC - TAIL_START, windows)
        flush()

    return gather_kernel


_gather = _make_gather()


def kernel(x, embedding_location):
    idx = x[:, 0].astype(jnp.int32)
    tail = jnp.pad(embedding_location[TAIL_START:].T,
                   ((0, 0), (0, 128 - (NUM_LOC - TAIL_START))))
    outp = _gather(idx, embedding_location.T, tail)
    return outp[:BATCH, :DIM]


# speedup vs baseline: 1.2156x; 1.1541x over previous
import functools

import jax
import jax.numpy as jnp
from jax import lax
from jax.experimental import pallas as pl
from jax.experimental.pallas import tpu as pltpu
from jax.experimental.pallas import tpu_sc as plsc

BATCH = 16384
DIM = 64
NUM_CORES = 2
NUM_SUBCORES = 16
NUM_WORKERS = NUM_CORES * NUM_SUBCORES

NUM_LOC = 1000000
TAIL_START = (NUM_LOC // 128) * 128
TC_PER_W = 245
LOC_PER_W = TC_PER_W * 128
WIN_TC = 4
WIN = WIN_TC * 128
N_WIN = 62
MAX_START_TC = (TAIL_START // 128) - WIN_TC

POS_BITS = 14
PACK = 1 << POS_BITS
SENTINEL = 0x7FFFFFFF

DUMP_ROW = BATCH
OUT_ROWS = BATCH + 128
FLUSH_AT = 108


def _make_gather():
    mesh = plsc.VectorSubcoreMesh(core_axis_name="c", subcore_axis_name="s")

    @functools.partial(
        pl.kernel,
        mesh=mesh,
        out_type=jax.ShapeDtypeStruct((OUT_ROWS, 128), jnp.float32),
        scratch_types=[
            pltpu.VMEM((BATCH,), jnp.int32),
            pltpu.VMEM((BATCH + 32,), jnp.int32),
            pltpu.VMEM((2, DIM, WIN), jnp.float32),
            pltpu.VMEM((DIM, 128), jnp.float32),
            pltpu.VMEM((128, 128), jnp.float32),
            pltpu.VMEM((1, 128), jnp.int32),
            pltpu.SemaphoreType.DMA,
            pltpu.SemaphoreType.DMA,
            pltpu.SemaphoreType.DMA,
        ],
        compiler_params=pltpu.CompilerParams(needs_layout_passes=False),
    )
    def gather_kernel(idx_hbm, table_hbm, tail_hbm, out_hbm,
                      idx_v, my_pk, win_v, tail_v, stage, posrow,
                      sem0, sem1, semw):
        wid = lax.axis_index("s") * NUM_CORES + lax.axis_index("c")
        lanes = jnp.arange(16, dtype=jnp.int32)

        pltpu.sync_copy(idx_hbm, idx_v)
        pltpu.sync_copy(tail_hbm, tail_v)

        lo_t = wid * LOC_PER_W
        hi_t = jnp.minimum(lo_t + LOC_PER_W, NUM_LOC)

        @pl.loop(0, BATCH // 16, init_carry=jnp.int32(0))
        def build(g, off):
            v = idx_v[pl.ds(g * 16, 16)]
            m = (v >= lo_t) & (v < hi_t)
            n = jnp.sum(m.astype(jnp.int32))

            @pl.when(n > 0)
            def _():
                pk = (v - lo_t) * PACK + (lanes + g * 16)
                key = jnp.where(m, pk, jnp.int32(SENTINEL))
                sk, _unused = plsc.sort_key_val(key, key)
                my_pk[pl.ds(off, 16)] = sk

            return off + n

        n_my = build
        my_pk[pl.ds(n_my, 16)] = jnp.full((16,), SENTINEL, jnp.int32)
        n_grp = (n_my + 15) >> 4

        def reset_posrow():
            for k in range(8):
                posrow[0, pl.ds(k * 16, 16)] = jnp.full(
                    (16,), DUMP_ROW, jnp.int32)

        reset_posrow()

        def flush():
            pltpu.async_copy(stage, out_hbm.at[posrow.at[0]], semw).wait()
            reset_posrow()

        def extract(win_ref, rel_lo, width, fill):
            plo = rel_lo * PACK
            phi = (rel_lo + width) * PACK

            @pl.loop(0, n_grp, init_carry=fill)
            def scan(k, fill):
                pk = my_pk[pl.ds(k * 16, 16)]
                m = (pk >= plo) & (pk < phi)
                n = jnp.sum(m.astype(jnp.int32))

                @pl.when(n > 0)
                def _():
                    il = jnp.where(m, (pk >> POS_BITS) - rel_lo, 0)
                    p = pk & (PACK - 1)
                    raw = fill + plsc.cumsum(m.astype(jnp.int32)) - 1
                    slots = jnp.where(m, raw, 127)
                    p_safe = jnp.where(m, p, DUMP_ROW)
                    for c in range(DIM):
                        cc = jnp.full((16,), c, jnp.int32)
                        vals = plsc.load_gather(win_ref, [cc, il])
                        plsc.store_scatter(stage, [slots, cc], vals)
                    plsc.store_scatter(posrow,
                                       [jnp.zeros((16,), jnp.int32), slots],
                                       p_safe)

                fill = fill + n

                @pl.when(fill >= FLUSH_AT)
                def _():
                    flush()

                return jnp.where(fill >= FLUSH_AT, 0, fill)

            return scan

        def start_tc_of(w):
            return jnp.minimum(wid * TC_PER_W + w * WIN_TC, MAX_START_TC)

        def rel_lo_of(w):
            return start_tc_of(w) * 128 - lo_t

        def fetch(w, buf, sem):
            return pltpu.async_copy(
                table_hbm.at[:, pl.ds(start_tc_of(w) * 128, WIN)],
                win_v.at[buf], sem)

        fetch(0, 0, sem0).wait()
        fetch(1, 1, sem1)

        @pl.loop(0, N_WIN // 2, init_carry=jnp.int32(0))
        def windows(j, fill):
            fill = extract(win_v.at[0], rel_lo_of(j * 2), WIN, fill)
            pltpu.make_async_copy(
                table_hbm.at[:, pl.ds(0, WIN)], win_v.at[1], sem1).wait()
            fetch(jnp.minimum(j * 2 + 2, N_WIN - 1), 0, sem0)
            fill = extract(win_v.at[1], rel_lo_of(j * 2 + 1), WIN, fill)
            fetch(jnp.minimum(j * 2 + 3, N_WIN - 1), 1, sem1)
            pltpu.make_async_copy(
                table_hbm.at[:, pl.ds(0, WIN)], win_v.at[0], sem0).wait()
            return fill

        pltpu.make_async_copy(
            table_hbm.at[:, pl.ds(0, WIN)], win_v.at[1], sem1).wait()

        tail_rel = jnp.minimum(jnp.int32(TAIL_START) - lo_t,
                               jnp.int32(LOC_PER_W + PACK))
        fill = extract(tail_v, tail_rel, NUM_LOC - TAIL_START, windows)
        flush()

    return gather_kernel


_gather = _make_gather()


def kernel(x, embedding_location):
    idx = x[:, 0].astype(jnp.int32)
    tail = jnp.pad(embedding_location[TAIL_START:].T,
                   ((0, 0), (0, 128 - (NUM_LOC - TAIL_START))))
    outp = _gather(idx, embedding_location.T, tail)
    return outp[:BATCH, :DIM]


# speedup vs baseline: 1.2283x; 1.0105x over previous
import functools

import jax
import jax.numpy as jnp
from jax import lax
from jax.experimental import pallas as pl
from jax.experimental.pallas import tpu as pltpu
from jax.experimental.pallas import tpu_sc as plsc

BATCH = 16384
DIM = 64
NUM_CORES = 2
NUM_SUBCORES = 16
NUM_WORKERS = NUM_CORES * NUM_SUBCORES

NUM_LOC = 1000000
TAIL_START = (NUM_LOC // 128) * 128
TC_PER_W = 245
LOC_PER_W = TC_PER_W * 128
WIN_TC = 4
WIN = WIN_TC * 128
N_WIN = 62
MAX_START_TC = (TAIL_START // 128) - WIN_TC

POS_BITS = 14
PACK = 1 << POS_BITS
SENTINEL = 0x7FFFFFFF

DUMP_ROW = BATCH
OUT_ROWS = BATCH + 128
FLUSH_AT = 108


def _make_gather():
    mesh = plsc.VectorSubcoreMesh(core_axis_name="c", subcore_axis_name="s")

    @functools.partial(
        pl.kernel,
        mesh=mesh,
        out_type=jax.ShapeDtypeStruct((OUT_ROWS, 128), jnp.float32),
        scratch_types=[
            pltpu.VMEM((BATCH,), jnp.int32),
            pltpu.VMEM((BATCH + 32,), jnp.int32),
            pltpu.VMEM((2, DIM, WIN), jnp.float32),
            pltpu.VMEM((DIM, 128), jnp.float32),
            pltpu.VMEM((128, 128), jnp.float32),
            pltpu.VMEM((1, 128), jnp.int32),
            pltpu.SemaphoreType.DMA,
            pltpu.SemaphoreType.DMA,
            pltpu.SemaphoreType.DMA,
        ],
        compiler_params=pltpu.CompilerParams(needs_layout_passes=False),
    )
    def gather_kernel(idx_hbm, table_hbm, tail_hbm, out_hbm,
                      idx_v, my_pk, win_v, tail_v, stage, posrow,
                      sem0, sem1, semw):
        wid = lax.axis_index("s") * NUM_CORES + lax.axis_index("c")
        lanes = jnp.arange(16, dtype=jnp.int32)

        pltpu.sync_copy(idx_hbm, idx_v)
        pltpu.sync_copy(tail_hbm, tail_v)

        lo_t = wid * LOC_PER_W
        hi_t = jnp.minimum(lo_t + LOC_PER_W, NUM_LOC)

        @pl.loop(0, BATCH // 16, init_carry=jnp.int32(0), unroll=4)
        def build(g, off):
            v = idx_v[pl.ds(g * 16, 16)]
            m = (v >= lo_t) & (v < hi_t)
            n = plsc.all_reduce_population_count(m)[0]

            @pl.when(n > 0)
            def _():
                pk = (v - lo_t) * PACK + (lanes + g * 16)
                key = jnp.where(m, pk, jnp.int32(SENTINEL))
                sk, _unused = plsc.sort_key_val(key, key)
                my_pk[pl.ds(off, 16)] = sk

            return off + n

        n_my = build
        my_pk[pl.ds(n_my, 16)] = jnp.full((16,), SENTINEL, jnp.int32)
        my_pk[pl.ds(n_my + 16, 16)] = jnp.full((16,), SENTINEL, jnp.int32)
        n_grp2 = (n_my + 31) >> 5

        def reset_posrow():
            for k in range(8):
                posrow[0, pl.ds(k * 16, 16)] = jnp.full(
                    (16,), DUMP_ROW, jnp.int32)

        reset_posrow()

        def flush():
            pltpu.async_copy(stage, out_hbm.at[posrow.at[0]], semw).wait()
            reset_posrow()

        def extract(win_ref, rel_lo, width, fill):
            plo = rel_lo * PACK
            phi = (rel_lo + width) * PACK

            @pl.loop(0, n_grp2, init_carry=fill)
            def scan(k2, fill):
                for half in range(2):
                    k16 = k2 * 32 + half * 16
                    pk = my_pk[pl.ds(k16, 16)]
                    m = (pk >= plo) & (pk < phi)
                    n = plsc.all_reduce_population_count(m)[0]

                    def _body(pk=pk, m=m, fill=fill):
                        il = jnp.where(m, (pk >> POS_BITS) - rel_lo, 0)
                        p = pk & (PACK - 1)
                        raw = fill + plsc.cumsum(m.astype(jnp.int32)) - 1
                        slots = jnp.where(m, raw, 127)
                        p_safe = jnp.where(m, p, DUMP_ROW)
                        for c in range(DIM):
                            cc = jnp.full((16,), c, jnp.int32)
                            vals = plsc.load_gather(win_ref, [cc, il])
                            plsc.store_scatter(stage, [slots, cc], vals)
                        plsc.store_scatter(posrow,
                                           [jnp.zeros((16,), jnp.int32),
                                            slots],
                                           p_safe)

                    pl.when(n > 0)(_body)
                    fill = fill + n

                    @pl.when(fill >= FLUSH_AT)
                    def _():
                        flush()

                    fill = jnp.where(fill >= FLUSH_AT, 0, fill)
                return fill

            return scan

        def start_tc_of(w):
            return jnp.minimum(wid * TC_PER_W + w * WIN_TC, MAX_START_TC)

        def rel_lo_of(w):
            return start_tc_of(w) * 128 - lo_t

        def fetch(w, buf, sem):
            return pltpu.async_copy(
                table_hbm.at[:, pl.ds(start_tc_of(w) * 128, WIN)],
                win_v.at[buf], sem)

        fetch(0, 0, sem0).wait()
        fetch(1, 1, sem1)

        @pl.loop(0, N_WIN // 2, init_carry=jnp.int32(0))
        def windows(j, fill):
            fill = extract(win_v.at[0], rel_lo_of(j * 2), WIN, fill)
            pltpu.make_async_copy(
                table_hbm.at[:, pl.ds(0, WIN)], win_v.at[1], sem1).wait()
            fetch(jnp.minimum(j * 2 + 2, N_WIN - 1), 0, sem0)
            fill = extract(win_v.at[1], rel_lo_of(j * 2 + 1), WIN, fill)
            fetch(jnp.minimum(j * 2 + 3, N_WIN - 1), 1, sem1)
            pltpu.make_async_copy(
                table_hbm.at[:, pl.ds(0, WIN)], win_v.at[0], sem0).wait()
            return fill

        pltpu.make_async_copy(
            table_hbm.at[:, pl.ds(0, WIN)], win_v.at[1], sem1).wait()

        tail_rel = jnp.minimum(jnp.int32(TAIL_START) - lo_t,
                               jnp.int32(LOC_PER_W + PACK))
        fill = extract(tail_v, tail_rel, NUM_LOC - TAIL_START, windows)
        flush()

    return gather_kernel


_gather = _make_gather()


def kernel(x, embedding_location):
    idx = x[:, 0].astype(jnp.int32)
    tail = jnp.pad(embedding_location[TAIL_START:].T,
                   ((0, 0), (0, 128 - (NUM_LOC - TAIL_START))))
    outp = _gather(idx, embedding_location.T, tail)
    return outp[:BATCH, :DIM]
